# baseline (device time: 52240 ns/iter reference)
import jax
import jax.numpy as jnp
from jax import lax
from jax.experimental import pallas as pl
from jax.experimental.pallas import tpu as pltpu

N_DEV = 32
BITS_A = (1, 2, 4, 8, 16)
BITS_B = (4, 8, 16, 1, 2)
NSTAGE = 5
V_PER = 4096
N_IDX = 1024
H = N_IDX // 2
D = 512
CHUNK = 1024


def kernel(table, idx):
    def body(table_ref, idx_ref, out_ref, recv_a, recv_b, send_sems, recv_sems):
        me = lax.axis_index("i")

        barrier_sem = pltpu.get_barrier_semaphore()
        for b in BITS_A:
            pl.semaphore_signal(
                barrier_sem, inc=1,
                device_id=(me ^ b,), device_id_type=pl.DeviceIdType.MESH,
            )
        pl.semaphore_wait(barrier_sem, NSTAGE)

        local = idx_ref[:] - me * V_PER
        local2d = local.reshape(N_IDX, 1)
        acc = jnp.zeros((N_IDX, D), jnp.float32)
        for j in range(V_PER // CHUNK):
            cols = lax.broadcasted_iota(jnp.int32, (N_IDX, CHUNK), 1) + j * CHUNK
            onehot = (cols == local2d).astype(jnp.bfloat16)
            t_chunk = table_ref[j * CHUNK:(j + 1) * CHUNK, :].astype(jnp.bfloat16)
            acc = acc + jnp.dot(onehot, t_chunk,
                                preferred_element_type=jnp.float32)
        out_ref[...] = acc.astype(jnp.bfloat16)

        def make_rs(bits, base, lo, k, recv, sem_off):
            half = (H // 2) >> k
            b = bits[k]
            keep_off = jnp.where((me & b) != 0, half, 0)
            give_lo = base + lo + (half - keep_off)
            rdma = pltpu.make_async_remote_copy(
                src_ref=out_ref.at[pl.ds(give_lo, half), :],
                dst_ref=recv.at[k, pl.ds(0, half), :],
                send_sem=send_sems.at[sem_off + k],
                recv_sem=recv_sems.at[sem_off + k],
                device_id=(me ^ b,),
                device_id_type=pl.DeviceIdType.MESH,
            )
            return rdma, keep_off, half

        lo_a = me * 0
        lo_b = me * 0
        los_a = []
        los_b = []
        for k in range(NSTAGE):
            los_a.append(lo_a)
            los_b.append(lo_b)
            rdma_a, keep_a, half = make_rs(BITS_A, 0, lo_a, k, recv_a, 0)
            rdma_a.start()
            rdma_b, keep_b, _ = make_rs(BITS_B, H, lo_b, k, recv_b, NSTAGE)
            rdma_b.start()
            rdma_a.wait()
            lo_a = lo_a + keep_a
            out_ref[pl.ds(lo_a, half), :] = (
                out_ref[pl.ds(lo_a, half), :] + recv_a[k, :half, :]
            )
            rdma_b.wait()
            lo_b = lo_b + keep_b
            out_ref[pl.ds(H + lo_b, half), :] = (
                out_ref[pl.ds(H + lo_b, half), :] + recv_b[k, :half, :]
            )

        def make_ag(bits, base, lo, k, sem_j):
            size = (H // 2) >> k
            rdma = pltpu.make_async_remote_copy(
                src_ref=out_ref.at[pl.ds(base + lo, size), :],
                dst_ref=out_ref.at[pl.ds(base + lo, size), :],
                send_sem=send_sems.at[sem_j],
                recv_sem=recv_sems.at[sem_j],
                device_id=(me ^ bits[k],),
                device_id_type=pl.DeviceIdType.MESH,
            )
            return rdma

        for k in reversed(range(NSTAGE)):
            j = NSTAGE - 1 - k
            rdma_a = make_ag(BITS_A, 0, lo_a, k, 2 * NSTAGE + j)
            rdma_a.start()
            rdma_b = make_ag(BITS_B, H, lo_b, k, 3 * NSTAGE + j)
            rdma_b.start()
            rdma_a.wait()
            lo_a = los_a[k]
            rdma_b.wait()
            lo_b = los_b[k]

    return pl.pallas_call(
        body,
        out_shape=jax.ShapeDtypeStruct((N_IDX, D), jnp.bfloat16),
        in_specs=[
            pl.BlockSpec(memory_space=pltpu.VMEM),
            pl.BlockSpec(memory_space=pltpu.VMEM),
        ],
        out_specs=pl.BlockSpec(memory_space=pltpu.VMEM),
        scratch_shapes=[
            pltpu.VMEM((NSTAGE, H // 2, D), jnp.bfloat16),
            pltpu.VMEM((NSTAGE, H // 2, D), jnp.bfloat16),
            pltpu.SemaphoreType.DMA((4 * NSTAGE,)),
            pltpu.SemaphoreType.DMA((4 * NSTAGE,)),
        ],
        compiler_params=pltpu.CompilerParams(collective_id=0),
    )(table, idx)


# device time: 45177 ns/iter; 1.1563x vs baseline; 1.1563x over previous
import jax
import jax.numpy as jnp
from jax import lax
from jax.experimental import pallas as pl
from jax.experimental.pallas import tpu as pltpu

N_DEV = 32
V_PER = 4096
N_IDX = 1024
D = 512
B = N_IDX // N_DEV
CHUNK = 1024


def kernel(table, idx):
    def body(table_ref, idx_ref, out_ref, gather_buf,
             send1, recv1, send2, recv2):
        me = lax.axis_index("i")

        barrier_sem = pltpu.get_barrier_semaphore()
        for d in range(1, N_DEV):
            pl.semaphore_signal(
                barrier_sem, inc=1,
                device_id=((me + d) % N_DEV,),
                device_id_type=pl.DeviceIdType.MESH,
            )
        pl.semaphore_wait(barrier_sem, N_DEV - 1)

        local = idx_ref[:] - me * V_PER
        local2d = local.reshape(N_IDX, 1)
        acc = jnp.zeros((N_IDX, D), jnp.float32)
        for j in range(V_PER // CHUNK):
            cols = lax.broadcasted_iota(jnp.int32, (N_IDX, CHUNK), 1) + j * CHUNK
            onehot = (cols == local2d).astype(jnp.bfloat16)
            t_chunk = table_ref[j * CHUNK:(j + 1) * CHUNK, :].astype(jnp.bfloat16)
            acc = acc + jnp.dot(onehot, t_chunk,
                                preferred_element_type=jnp.float32)
        out_ref[...] = acc.astype(jnp.bfloat16)

        p1 = []
        for d in range(1, N_DEV):
            p = (me + d) % N_DEV
            rdma = pltpu.make_async_remote_copy(
                src_ref=out_ref.at[pl.ds(p * B, B), :],
                dst_ref=gather_buf.at[pl.ds(d * B, B), :],
                send_sem=send1.at[d],
                recv_sem=recv1.at[d],
                device_id=(p,),
                device_id_type=pl.DeviceIdType.MESH,
            )
            rdma.start()
            p1.append(rdma)

        gather_buf[pl.ds(0, B), :] = out_ref[pl.ds(me * B, B), :]
        for d in range(1, N_DEV):
            p1[d - 1].wait_recv()
        blk = gather_buf[pl.ds(0, B), :]
        for d in range(1, N_DEV):
            blk = blk + gather_buf[d * B:(d + 1) * B, :]
        out_ref[pl.ds(me * B, B), :] = blk

        p2 = []
        for d in range(1, N_DEV):
            rdma = pltpu.make_async_remote_copy(
                src_ref=out_ref.at[pl.ds(me * B, B), :],
                dst_ref=out_ref.at[pl.ds(me * B, B), :],
                send_sem=send2.at[d],
                recv_sem=recv2.at[d],
                device_id=((me + d) % N_DEV,),
                device_id_type=pl.DeviceIdType.MESH,
            )
            rdma.start()
            p2.append(rdma)

        for d in range(1, N_DEV):
            p1[d - 1].wait_send()
        for d in range(1, N_DEV):
            p2[d - 1].wait()

    return pl.pallas_call(
        body,
        out_shape=jax.ShapeDtypeStruct((N_IDX, D), jnp.bfloat16),
        in_specs=[
            pl.BlockSpec(memory_space=pltpu.VMEM),
            pl.BlockSpec(memory_space=pltpu.VMEM),
        ],
        out_specs=pl.BlockSpec(memory_space=pltpu.VMEM),
        scratch_shapes=[
            pltpu.VMEM((N_DEV * B, D), jnp.bfloat16),
            pltpu.SemaphoreType.DMA((N_DEV,)),
            pltpu.SemaphoreType.DMA((N_DEV,)),
            pltpu.SemaphoreType.DMA((N_DEV,)),
            pltpu.SemaphoreType.DMA((N_DEV,)),
        ],
        compiler_params=pltpu.CompilerParams(collective_id=0),
    )(table, idx)
